# baseline (speedup 1.0000x reference)
import functools

import jax
import jax.numpy as jnp
from jax import lax
from jax.experimental import pallas as pl
from jax.experimental.pallas import tpu as pltpu
from jax.experimental.pallas import tpu_sc as plsc

B, F, D = 4096, 26, 16
SEG = 100000
SEGP = SEG + 96
TOTAL = F * SEG
NW = 32
FH = F // 2
FDH = FH * D
H1, H2 = 256, 128
BLK = 512
NT = FDH + FH
KMAX = -(-NT // NW)


def _sc_gather_half(xt, emb_t, lin_t, f0):
    mesh = plsc.VectorSubcoreMesh(core_axis_name="c", subcore_axis_name="s")

    @functools.partial(
        pl.kernel,
        mesh=mesh,
        out_type=[
            jax.ShapeDtypeStruct((FDH, B), jnp.float32),
            jax.ShapeDtypeStruct((FH, B), jnp.float32),
        ],
        scratch_types=[
            pltpu.VMEM((B,), jnp.int32),
            pltpu.VMEM((SEGP,), jnp.float32),
            pltpu.VMEM((B,), jnp.float32),
        ],
        compiler_params=pltpu.CompilerParams(needs_layout_passes=False),
    )
    def k(xt_hbm, emb_hbm, lin_hbm, emb_out, lin_out, ids_v, seg_v, out_v):
        wid = lax.axis_index("s") * 2 + lax.axis_index("c")

        def pick_all(shift):
            def body(i, _):
                idx = ids_v[pl.ds(i * 16, 16)] + shift
                out_v[pl.ds(i * 16, 16)] = plsc.load_gather(seg_v, [idx])
                return 0
            lax.fori_loop(0, B // 16, body, 0)

        def seg_start(f):
            a = f * SEG
            sa = pl.multiple_of(a - lax.rem(a, 128), 128)
            return sa, a - sa

        for j in range(KMAX):
            t = wid + NW * j

            @pl.when(t < NT)
            def _():
                is_emb = t < FDH
                fl = jnp.where(is_emb, t // D, t - FDH)
                d = lax.rem(t, D)
                sa, shift = seg_start(f0 + fl)
                pltpu.sync_copy(xt_hbm.at[f0 + fl], ids_v)

                @pl.when(is_emb)
                def _():
                    pltpu.sync_copy(emb_hbm.at[d, pl.ds(sa, SEGP)], seg_v)
                    pick_all(shift)
                    pltpu.sync_copy(out_v, emb_out.at[t])

                @pl.when(jnp.logical_not(is_emb))
                def _():
                    pltpu.sync_copy(lin_hbm.at[0, pl.ds(sa, SEGP)], seg_v)
                    pick_all(shift)
                    pltpu.sync_copy(out_v, lin_out.at[t - FDH])

    return k(xt, emb_t, lin_t)


def _tc_part_body(emb_ref, lin_ref, W1_ref, out_h, out_se, out_sc):
    et = emb_ref[...]
    r = lax.broadcasted_iota(jnp.int32, (D, FDH), 1)
    c = lax.broadcasted_iota(jnp.int32, (D, FDH), 0)
    sel = jnp.where((r % D) == c, 1.0, 0.0)
    out_se[...] = jnp.dot(sel, et, preferred_element_type=jnp.float32)
    t2 = jnp.sum(et * et, axis=0, keepdims=True)
    first = jnp.sum(lin_ref[...], axis=0, keepdims=True)
    out_sc[...] = jnp.concatenate([t2] * 4 + [first] * 4, axis=0)
    dn = (((0,), (0,)), ((), ()))
    out_h[...] = lax.dot_general(W1_ref[...], et, dn,
                                 preferred_element_type=jnp.float32)


def _tc_partial(emb_h, lin_h, W1h):
    grid = (B // BLK,)
    full = lambda shape: pl.BlockSpec(shape, lambda i: (0, 0))
    return pl.pallas_call(
        _tc_part_body,
        grid=grid,
        in_specs=[
            pl.BlockSpec((FDH, BLK), lambda i: (0, i)),
            pl.BlockSpec((FH, BLK), lambda i: (0, i)),
            full((FDH, H1)),
        ],
        out_specs=[
            pl.BlockSpec((H1, BLK), lambda i: (0, i)),
            pl.BlockSpec((D, BLK), lambda i: (0, i)),
            pl.BlockSpec((8, BLK), lambda i: (0, i)),
        ],
        out_shape=[
            jax.ShapeDtypeStruct((H1, B), jnp.float32),
            jax.ShapeDtypeStruct((D, B), jnp.float32),
            jax.ShapeDtypeStruct((8, B), jnp.float32),
        ],
    )(emb_h, lin_h, W1h)


def _tc_final_body(emb_ref, lin_ref, W1_ref, hp_ref, se_ref, sc_ref,
                   s1_ref, t1_ref, W2_ref, s2_ref, t2_ref, w3_ref, c_ref,
                   out_ref):
    et = emb_ref[...]
    r = lax.broadcasted_iota(jnp.int32, (D, FDH), 1)
    c = lax.broadcasted_iota(jnp.int32, (D, FDH), 0)
    sel = jnp.where((r % D) == c, 1.0, 0.0)
    sum_e = se_ref[...] + jnp.dot(sel, et, preferred_element_type=jnp.float32)
    t1 = jnp.sum(sum_e * sum_e, axis=0, keepdims=True)
    t2 = sc_ref[0:1, :] + jnp.sum(et * et, axis=0, keepdims=True)
    second = 0.5 * (t1 - t2)
    first = sc_ref[4:5, :] + jnp.sum(lin_ref[...], axis=0, keepdims=True)
    dn = (((0,), (0,)), ((), ()))
    h = hp_ref[...] + lax.dot_general(W1_ref[...], et, dn,
                                      preferred_element_type=jnp.float32)
    h = jnp.maximum(h * s1_ref[...] + t1_ref[...], 0.0)
    h = lax.dot_general(W2_ref[...], h, dn,
                        preferred_element_type=jnp.float32)
    h = jnp.maximum(h * s2_ref[...] + t2_ref[...], 0.0)
    deep = lax.dot_general(w3_ref[...], h, dn,
                           preferred_element_type=jnp.float32)
    out_ref[...] = first + second + deep + c_ref[0, 0]


def _tc_final(emb_h, lin_h, W1h, hp, se, sc, s1, t1, W2, s2, t2, w3, cbias):
    grid = (B // BLK,)
    full = lambda shape: pl.BlockSpec(shape, lambda i: (0, 0))
    col = lambda shape: pl.BlockSpec(shape, lambda i: (0, i))
    return pl.pallas_call(
        _tc_final_body,
        grid=grid,
        in_specs=[
            col((FDH, BLK)),
            col((FH, BLK)),
            full((FDH, H1)),
            col((H1, BLK)),
            col((D, BLK)),
            col((8, BLK)),
            full((H1, 1)),
            full((H1, 1)),
            full((H1, H2)),
            full((H2, 1)),
            full((H2, 1)),
            full((H2, 1)),
            full((1, 1)),
        ],
        out_specs=pl.BlockSpec((1, BLK), lambda i: (0, i)),
        out_shape=jax.ShapeDtypeStruct((1, B), jnp.float32),
    )(emb_h, lin_h, W1h, hp, se, sc, s1, t1, W2, s2, t2, w3, cbias)


def kernel(x, lin_w, lin_b, emb_w, W1, b1, g1, be1, W2, b2, g2, be2, W3, b3):
    xt = x.T
    emb_t = emb_w.T
    lin_t = lin_w.T

    emb_a, lin_a = _sc_gather_half(xt, emb_t, lin_t, 0)
    emb_b, lin_b2 = _sc_gather_half(xt, emb_t, lin_t, FH)

    W1a = W1[:FDH]
    W1b = W1[FDH:]
    hp, se, sc = _tc_partial(emb_a, lin_a, W1a)

    inv = 1.0 / jnp.sqrt(jnp.float32(1.0 + 1e-5))
    s1 = (g1 * inv).reshape(H1, 1)
    t1 = (b1 * g1 * inv + be1).reshape(H1, 1)
    s2 = (g2 * inv).reshape(H2, 1)
    t2 = (b2 * g2 * inv + be2).reshape(H2, 1)
    cbias = (lin_b + b3).reshape(1, 1)

    out = _tc_final(emb_b, lin_b2, W1b, hp, se, sc, s1, t1, W2, s2, t2, W3,
                    cbias)
    return out.reshape(B)


# speedup vs baseline: 1.0748x; 1.0748x over previous
import functools

import jax
import jax.numpy as jnp
from jax import lax
from jax.experimental import pallas as pl
from jax.experimental.pallas import tpu as pltpu
from jax.experimental.pallas import tpu_sc as plsc

B, F, D = 4096, 26, 16
SEG = 100000
SEGP = SEG + 96
TOTAL = F * SEG
NW = 32
FD = F * D
TPW = FD // NW
H1, H2 = 256, 128
BLK = 1024


def _sc_gather(xt, emb_t, lin_t):
    mesh = plsc.VectorSubcoreMesh(core_axis_name="c", subcore_axis_name="s")

    @functools.partial(
        pl.kernel,
        mesh=mesh,
        out_type=[
            jax.ShapeDtypeStruct((FD, B), jnp.float32),
            jax.ShapeDtypeStruct((F, B), jnp.float32),
        ],
        scratch_types=[
            pltpu.VMEM((B,), jnp.int32),
            pltpu.VMEM((SEGP,), jnp.float32),
            pltpu.VMEM((B,), jnp.float32),
        ],
        compiler_params=pltpu.CompilerParams(needs_layout_passes=False),
    )
    def k(xt_hbm, emb_hbm, lin_hbm, emb_out, lin_out, ids_v, seg_v, out_v):
        wid = lax.axis_index("s") * 2 + lax.axis_index("c")

        def pick_all(shift):
            def body(i, _):
                idx = ids_v[pl.ds(i * 16, 16)] + shift
                out_v[pl.ds(i * 16, 16)] = plsc.load_gather(seg_v, [idx])
                return 0
            lax.fori_loop(0, B // 16, body, 0)

        def seg_start(f):
            a = f * SEG
            sa = pl.multiple_of(a - lax.rem(a, 128), 128)
            return sa, a - sa

        for j in range(TPW):
            t = wid * TPW + j
            f = t // D
            d = t % D
            sa, shift = seg_start(f)
            pltpu.sync_copy(xt_hbm.at[f], ids_v)
            pltpu.sync_copy(emb_hbm.at[d, pl.ds(sa, SEGP)], seg_v)
            pick_all(shift)
            pltpu.sync_copy(out_v, emb_out.at[t])

        @pl.when(wid < F)
        def _():
            sa, shift = seg_start(wid)
            pltpu.sync_copy(xt_hbm.at[wid], ids_v)
            pltpu.sync_copy(lin_hbm.at[0, pl.ds(sa, SEGP)], seg_v)
            pick_all(shift)
            pltpu.sync_copy(out_v, lin_out.at[wid])

    return k(xt, emb_t, lin_t)


def _tc_body(emb_ref, lin_ref, W1_ref, s1_ref, t1_ref, W2_ref, s2_ref,
             t2_ref, w3_ref, cb_ref, out_ref):
    et = emb_ref[...]
    r = lax.broadcasted_iota(jnp.int32, (D, FD), 1)
    c = lax.broadcasted_iota(jnp.int32, (D, FD), 0)
    sel = jnp.where((r % D) == c, 1.0, 0.0)
    sum_e = jnp.dot(sel, et, preferred_element_type=jnp.float32)
    t1 = jnp.sum(sum_e * sum_e, axis=0, keepdims=True)
    t2 = jnp.sum(et * et, axis=0, keepdims=True)
    second = 0.5 * (t1 - t2)
    first = jnp.sum(lin_ref[...], axis=0, keepdims=True)
    dn = (((0,), (0,)), ((), ()))
    h = lax.dot_general(W1_ref[...], et, dn,
                        preferred_element_type=jnp.float32)
    h = jnp.maximum(h * s1_ref[...] + t1_ref[...], 0.0)
    h = lax.dot_general(W2_ref[...], h, dn,
                        preferred_element_type=jnp.float32)
    h = jnp.maximum(h * s2_ref[...] + t2_ref[...], 0.0)
    deep = lax.dot_general(w3_ref[...], h, dn,
                           preferred_element_type=jnp.float32)
    out_ref[...] = first + second + deep + cb_ref[0, 0]


def _tc_dense(emb_t, lin_t, W1, s1, t1, W2, s2, t2, w3, cb):
    grid = (B // BLK,)
    full = lambda shape: pl.BlockSpec(shape, lambda i: (0, 0))
    return pl.pallas_call(
        _tc_body,
        grid=grid,
        in_specs=[
            pl.BlockSpec((FD, BLK), lambda i: (0, i)),
            pl.BlockSpec((F, BLK), lambda i: (0, i)),
            full((FD, H1)),
            full((H1, 1)),
            full((H1, 1)),
            full((H1, H2)),
            full((H2, 1)),
            full((H2, 1)),
            full((H2, 1)),
            full((1, 1)),
        ],
        out_specs=pl.BlockSpec((1, BLK), lambda i: (0, i)),
        out_shape=jax.ShapeDtypeStruct((1, B), jnp.float32),
    )(emb_t, lin_t, W1, s1, t1, W2, s2, t2, w3, cb)


def kernel(x, lin_w, lin_b, emb_w, W1, b1, g1, be1, W2, b2, g2, be2, W3, b3):
    xt = x.T
    emb_t = emb_w.T
    lin_t = lin_w.T

    emb_feat, lin_feat = _sc_gather(xt, emb_t, lin_t)

    inv = 1.0 / jnp.sqrt(jnp.float32(1.0 + 1e-5))
    s1 = (g1 * inv).reshape(H1, 1)
    t1 = (b1 * g1 * inv + be1).reshape(H1, 1)
    s2 = (g2 * inv).reshape(H2, 1)
    t2 = (b2 * g2 * inv + be2).reshape(H2, 1)
    w3 = W3
    cb = (lin_b + b3).reshape(1, 1)

    out = _tc_dense(emb_feat, lin_feat, W1, s1, t1, W2, s2, t2, w3, cb)
    return out.reshape(B)


# speedup vs baseline: 1.0895x; 1.0137x over previous
import functools

import jax
import jax.numpy as jnp
from jax import lax
from jax.experimental import pallas as pl
from jax.experimental.pallas import tpu as pltpu
from jax.experimental.pallas import tpu_sc as plsc

B, F, D = 4096, 26, 16
SEG = 100000
SEGP = SEG + 96
TOTAL = F * SEG
NW = 32
FD = F * D
TPW = FD // NW
H1, H2 = 256, 128
BLK = 2048


def _sc_gather(xt, emb_t, lin_t):
    mesh = plsc.VectorSubcoreMesh(core_axis_name="c", subcore_axis_name="s")

    @functools.partial(
        pl.kernel,
        mesh=mesh,
        out_type=[
            jax.ShapeDtypeStruct((FD, B), jnp.float32),
            jax.ShapeDtypeStruct((F, B), jnp.float32),
        ],
        scratch_types=[
            pltpu.VMEM((B,), jnp.int32),
            pltpu.VMEM((SEGP,), jnp.float32),
            pltpu.VMEM((B,), jnp.float32),
        ],
        compiler_params=pltpu.CompilerParams(needs_layout_passes=False),
    )
    def k(xt_hbm, emb_hbm, lin_hbm, emb_out, lin_out, ids_v, seg_v, out_v):
        wid = lax.axis_index("s") * 2 + lax.axis_index("c")

        def pick_all(shift):
            def body(i, _):
                idx = ids_v[pl.ds(i * 16, 16)] + shift
                out_v[pl.ds(i * 16, 16)] = plsc.load_gather(seg_v, [idx])
                return 0
            lax.fori_loop(0, B // 16, body, 0)

        def seg_start(f):
            a = f * SEG
            sa = pl.multiple_of(a - lax.rem(a, 128), 128)
            return sa, a - sa

        for j in range(TPW):
            t = wid * TPW + j
            f = t // D
            d = t % D
            sa, shift = seg_start(f)
            pltpu.sync_copy(xt_hbm.at[f], ids_v)
            pltpu.sync_copy(emb_hbm.at[d, pl.ds(sa, SEGP)], seg_v)
            pick_all(shift)
            pltpu.sync_copy(out_v, emb_out.at[t])

        @pl.when(wid < F)
        def _():
            sa, shift = seg_start(wid)
            pltpu.sync_copy(xt_hbm.at[wid], ids_v)
            pltpu.sync_copy(lin_hbm.at[0, pl.ds(sa, SEGP)], seg_v)
            pick_all(shift)
            pltpu.sync_copy(out_v, lin_out.at[wid])

    return k(xt, emb_t, lin_t)


def _tc_body(emb_ref, lin_ref, W1_ref, s1_ref, t1_ref, W2_ref, s2_ref,
             t2_ref, w3_ref, cb_ref, out_ref):
    et = emb_ref[...]
    r = lax.broadcasted_iota(jnp.int32, (D, FD), 1)
    c = lax.broadcasted_iota(jnp.int32, (D, FD), 0)
    sel = jnp.where((r % D) == c, 1.0, 0.0)
    sum_e = jnp.dot(sel, et, preferred_element_type=jnp.float32)
    t1 = jnp.sum(sum_e * sum_e, axis=0, keepdims=True)
    t2 = jnp.sum(et * et, axis=0, keepdims=True)
    second = 0.5 * (t1 - t2)
    first = jnp.sum(lin_ref[...], axis=0, keepdims=True)
    dn = (((0,), (0,)), ((), ()))
    h = lax.dot_general(W1_ref[...], et, dn,
                        preferred_element_type=jnp.float32)
    h = jnp.maximum(h * s1_ref[...] + t1_ref[...], 0.0)
    h = lax.dot_general(W2_ref[...], h, dn,
                        preferred_element_type=jnp.float32)
    h = jnp.maximum(h * s2_ref[...] + t2_ref[...], 0.0)
    deep = lax.dot_general(w3_ref[...], h, dn,
                           preferred_element_type=jnp.float32)
    out_ref[...] = first + second + deep + cb_ref[0, 0]


def _tc_dense(emb_t, lin_t, W1, s1, t1, W2, s2, t2, w3, cb):
    grid = (B // BLK,)
    full = lambda shape: pl.BlockSpec(shape, lambda i: (0, 0))
    return pl.pallas_call(
        _tc_body,
        grid=grid,
        in_specs=[
            pl.BlockSpec((FD, BLK), lambda i: (0, i)),
            pl.BlockSpec((F, BLK), lambda i: (0, i)),
            full((FD, H1)),
            full((H1, 1)),
            full((H1, 1)),
            full((H1, H2)),
            full((H2, 1)),
            full((H2, 1)),
            full((H2, 1)),
            full((1, 1)),
        ],
        out_specs=pl.BlockSpec((1, BLK), lambda i: (0, i)),
        out_shape=jax.ShapeDtypeStruct((1, B), jnp.float32),
    )(emb_t, lin_t, W1, s1, t1, W2, s2, t2, w3, cb)


def kernel(x, lin_w, lin_b, emb_w, W1, b1, g1, be1, W2, b2, g2, be2, W3, b3):
    xt = x.T
    emb_t = emb_w.T
    lin_t = lin_w.T

    emb_feat, lin_feat = _sc_gather(xt, emb_t, lin_t)

    inv = 1.0 / jnp.sqrt(jnp.float32(1.0 + 1e-5))
    s1 = (g1 * inv).reshape(H1, 1)
    t1 = (b1 * g1 * inv + be1).reshape(H1, 1)
    s2 = (g2 * inv).reshape(H2, 1)
    t2 = (b2 * g2 * inv + be2).reshape(H2, 1)
    w3 = W3
    cb = (lin_b + b3).reshape(1, 1)

    out = _tc_dense(emb_feat, lin_feat, W1, s1, t1, W2, s2, t2, w3, cb)
    return out.reshape(B)
